# Initial kernel scaffold; baseline (speedup 1.0000x reference)
import jax
import jax.numpy as jnp
from jax.experimental import pallas as pl


def kernel(x, edge_index, edge_attr, batch, W, att_src, att_dst, b_gat, out_W, out_b):
    raise NotImplementedError("write your pallas kernel here")


# speedup vs baseline: 33.2631x; 33.2631x over previous
import functools

import jax
import jax.numpy as jnp
from jax import lax
from jax.experimental import pallas as pl
from jax.experimental.pallas import tpu as pltpu
from jax.experimental.pallas import tpu_sc as plsc

N = 10000
E = 320000
F_IN = 128
F_OUT = 20
G = 64
N_OUT = 5
FP = 32
NC = 2
NS = 16
EPT = E // (NC * NS)
K = 80
CHUNKS = EPT // K
RPT = 632
RLAST = N - (NS - 1) * RPT


def _prep_body(x_ref, w_ref, asw_ref, adw_ref, hpad_ref, asrc_ref, adst_ref,
               c_ref):
    h = jnp.dot(x_ref[...], w_ref[...], preferred_element_type=jnp.float32)
    hpad_ref[...] = jnp.concatenate(
        [h, jnp.ones((N, 1), jnp.float32),
         jnp.zeros((N, FP - F_OUT - 1), jnp.float32)], axis=1)
    asrc = jnp.sum(h * asw_ref[...], axis=1, keepdims=True)
    adst = jnp.sum(h * adw_ref[...], axis=1, keepdims=True)
    asrc_ref[...] = asrc
    adst_ref[...] = adst
    ub = jnp.max(asrc) + jnp.max(adst)
    c = jnp.where(ub >= 0.0, ub, 0.2 * ub)
    c_ref[...] = jnp.full((1, 128), c, jnp.float32)


_prep = pl.pallas_call(
    _prep_body,
    out_shape=[
        jax.ShapeDtypeStruct((N, FP), jnp.float32),
        jax.ShapeDtypeStruct((N, 1), jnp.float32),
        jax.ShapeDtypeStruct((N, 1), jnp.float32),
        jax.ShapeDtypeStruct((1, 128), jnp.float32),
    ],
)


_mesh = plsc.VectorSubcoreMesh(core_axis_name="c", subcore_axis_name="s",
                               num_cores=NC, num_subcores=NS)


@functools.partial(
    pl.kernel,
    mesh=_mesh,
    compiler_params=pltpu.CompilerParams(needs_layout_passes=False,
                                         use_tc_tiling_on_sc=False),
    out_type=jax.ShapeDtypeStruct((NC, N, FP), jnp.float32),
    scratch_types=[
        pltpu.VMEM((N,), jnp.float32),
        pltpu.VMEM((N,), jnp.float32),
        pltpu.VMEM((16,), jnp.float32),
        pltpu.VMEM((K,), jnp.int32),
        pltpu.VMEM((K,), jnp.int32),
        pltpu.VMEM((K, FP), jnp.float32),
        pltpu.VMEM((16,), jnp.float32),
        pltpu.VMEM((RPT, FP), jnp.float32),
        pltpu.VMEM_SHARED((N, FP), jnp.float32),
        pltpu.SemaphoreType.DMA,
    ],
)
def _edge_kernel(hpad_hbm, src_hbm, dst_hbm, asrc_hbm, adst_hbm, c_hbm,
                 out_hbm, asrc_v, adst_v, c_v, srcb, dstb, gbuf, exb, zbuf,
                 acc, sem):
    cid = lax.axis_index("c")
    sid = lax.axis_index("s")

    z16 = jnp.zeros((16,), jnp.float32)

    def _zrow(i, carry):
        zbuf[i, pl.ds(0, 16)] = z16
        zbuf[i, pl.ds(16, 16)] = z16
        return carry

    lax.fori_loop(0, RPT, _zrow, 0)

    @pl.when(sid < NS - 1)
    def _zero_main():
        pltpu.sync_copy(zbuf, acc.at[pl.ds(sid * RPT, RPT)])

    @pl.when(sid == NS - 1)
    def _zero_last():
        pltpu.sync_copy(zbuf.at[pl.ds(0, RLAST)],
                        acc.at[pl.ds((NS - 1) * RPT, RLAST)])

    pltpu.sync_copy(asrc_hbm, asrc_v)
    pltpu.sync_copy(adst_hbm, adst_v)
    pltpu.sync_copy(c_hbm, c_v)
    plsc.subcore_barrier()

    cvec = c_v[...]
    ebase = (cid * NS + sid) * EPT

    def _chunk(t, carry):
        off = ebase + t * K
        pltpu.sync_copy(src_hbm.at[pl.ds(off, K)], srcb)
        pltpu.sync_copy(dst_hbm.at[pl.ds(off, K)], dstb)
        pltpu.async_copy(hpad_hbm.at[srcb], gbuf, sem).wait()
        for g in range(K // 16):
            s16 = srcb[pl.ds(g * 16, 16)]
            d16 = dstb[pl.ds(g * 16, 16)]
            e = plsc.load_gather(asrc_v, [s16]) + plsc.load_gather(adst_v, [d16])
            e = jnp.where(e >= 0.0, e, 0.2 * e)
            exb[...] = jnp.exp(e - cvec)
            for i in range(16):
                j = g * 16 + i
                w = plsc.load_gather(exb, [jnp.full((16,), i, jnp.int32)])
                gbuf[j, pl.ds(0, 16)] = gbuf[j, pl.ds(0, 16)] * w
                gbuf[j, pl.ds(16, 16)] = gbuf[j, pl.ds(16, 16)] * w
        pltpu.sync_copy(gbuf, acc.at[dstb], add=True)
        return carry

    lax.fori_loop(0, CHUNKS, _chunk, 0)
    plsc.subcore_barrier()

    @pl.when(sid < NS - 1)
    def _out_main():
        pltpu.sync_copy(acc.at[pl.ds(sid * RPT, RPT)],
                        out_hbm.at[cid, pl.ds(sid * RPT, RPT)])

    @pl.when(sid == NS - 1)
    def _out_last():
        pltpu.sync_copy(acc.at[pl.ds((NS - 1) * RPT, RLAST)],
                        out_hbm.at[cid, pl.ds((NS - 1) * RPT, RLAST)])


def _finish_body(p_ref, hpad_ref, asrc_ref, adst_ref, c_ref, batch_ref,
                 bgat_ref, outw_ref, outb_ref, o_ref):
    tot = p_ref[0] + p_ref[1]
    e = asrc_ref[...] + adst_ref[...]
    e = jnp.where(e >= 0.0, e, 0.2 * e)
    es = jnp.exp(e - c_ref[0:1, 0:1])
    tot = tot + es * hpad_ref[...]
    node = tot[:, :F_OUT] / (tot[:, F_OUT:F_OUT + 1] + 1e-16) + bgat_ref[...]
    node = jnp.where(node >= 0.0, node, 0.01 * node)
    gids = lax.broadcasted_iota(jnp.int32, (N, G), 1)
    onehot = (batch_ref[...] == gids).astype(jnp.float32)
    sums = lax.dot_general(onehot, node, (((0,), (0,)), ((), ())),
                           preferred_element_type=jnp.float32)
    cnt = lax.dot_general(onehot, jnp.ones((N, 1), jnp.float32),
                          (((0,), (0,)), ((), ())),
                          preferred_element_type=jnp.float32)
    pooled = sums / jnp.maximum(cnt, 1.0)
    logits = jnp.dot(pooled, outw_ref[...],
                     preferred_element_type=jnp.float32) + outb_ref[...]
    mx = jnp.max(logits, axis=1, keepdims=True)
    ex2 = jnp.exp(logits - mx)
    o_ref[...] = ex2 / jnp.sum(ex2, axis=1, keepdims=True)


_finish = pl.pallas_call(
    _finish_body,
    out_shape=jax.ShapeDtypeStruct((G, N_OUT), jnp.float32),
)


def kernel(x, edge_index, edge_attr, batch, W, att_src, att_dst, b_gat,
           out_W, out_b):
    del edge_attr
    hpad, asrc, adst, crow = _prep(x, W, att_src.reshape(1, F_OUT),
                                   att_dst.reshape(1, F_OUT))
    partials = _edge_kernel(hpad, edge_index[0], edge_index[1],
                            asrc.reshape(N), adst.reshape(N), crow[0, :16])
    return _finish(partials, hpad, asrc, adst, crow, batch.reshape(N, 1),
                   b_gat.reshape(1, F_OUT), out_W, out_b)


# speedup vs baseline: 66.9552x; 2.0129x over previous
import functools

import jax
import jax.numpy as jnp
from jax import lax
from jax.experimental import pallas as pl
from jax.experimental.pallas import tpu as pltpu
from jax.experimental.pallas import tpu_sc as plsc

N = 10000
E = 320000
F_IN = 128
F_OUT = 20
G = 64
N_OUT = 5
FP = 32
NC = 2
NS = 16
EPT = E // (NC * NS)
K = 80
CHUNKS = EPT // K
RPT = 632
RLAST = N - (NS - 1) * RPT


def _prep_body(x_ref, w_ref, asw_ref, adw_ref, hpad_ref, asrc_ref, adst_ref,
               c_ref):
    h = jnp.dot(x_ref[...], w_ref[...], preferred_element_type=jnp.float32)
    hpad_ref[...] = jnp.concatenate(
        [h, jnp.ones((N, 1), jnp.float32),
         jnp.zeros((N, FP - F_OUT - 1), jnp.float32)], axis=1)
    asrc = jnp.sum(h * asw_ref[...], axis=1, keepdims=True)
    adst = jnp.sum(h * adw_ref[...], axis=1, keepdims=True)
    asrc_ref[...] = asrc
    adst_ref[...] = adst
    ub = jnp.max(asrc) + jnp.max(adst)
    c = jnp.where(ub >= 0.0, ub, 0.2 * ub)
    c_ref[...] = jnp.full((1, 128), c, jnp.float32)


_prep = pl.pallas_call(
    _prep_body,
    out_shape=[
        jax.ShapeDtypeStruct((N, FP), jnp.float32),
        jax.ShapeDtypeStruct((N, 1), jnp.float32),
        jax.ShapeDtypeStruct((N, 1), jnp.float32),
        jax.ShapeDtypeStruct((1, 128), jnp.float32),
    ],
)


_mesh = plsc.VectorSubcoreMesh(core_axis_name="c", subcore_axis_name="s",
                               num_cores=NC, num_subcores=NS)


@functools.partial(
    pl.kernel,
    mesh=_mesh,
    compiler_params=pltpu.CompilerParams(needs_layout_passes=False,
                                         use_tc_tiling_on_sc=False),
    out_type=jax.ShapeDtypeStruct((NC, N, FP), jnp.float32),
    scratch_types=[
        pltpu.VMEM((N,), jnp.float32),
        pltpu.VMEM((N,), jnp.float32),
        pltpu.VMEM((16,), jnp.float32),
        pltpu.VMEM((CHUNKS, K), jnp.int32),
        pltpu.VMEM((CHUNKS, K), jnp.int32),
        pltpu.VMEM((K, FP), jnp.float32),
        pltpu.VMEM((K, FP), jnp.float32),
        pltpu.VMEM((16,), jnp.float32),
        pltpu.VMEM((RPT, FP), jnp.float32),
        pltpu.VMEM_SHARED((N, FP), jnp.float32),
        pltpu.SemaphoreType.DMA,
        pltpu.SemaphoreType.DMA,
        pltpu.SemaphoreType.DMA,
        pltpu.SemaphoreType.DMA,
        pltpu.SemaphoreType.DMA,
    ],
)
def _edge_kernel(hpad_hbm, src_hbm, dst_hbm, asrc_hbm, adst_hbm, c_hbm,
                 out_hbm, asrc_v, adst_v, c_v, srcb, dstb, gbuf0, gbuf1, exb,
                 zbuf, acc, insem, gsem0, gsem1, ssem0, ssem1):
    cid = lax.axis_index("c")
    sid = lax.axis_index("s")
    rowbase = (cid * NS + sid) * CHUNKS

    pltpu.async_copy(asrc_hbm, asrc_v, insem)
    pltpu.async_copy(adst_hbm, adst_v, insem)
    pltpu.async_copy(c_hbm, c_v, insem)
    pltpu.async_copy(src_hbm.at[pl.ds(rowbase, CHUNKS)], srcb, insem)
    pltpu.async_copy(dst_hbm.at[pl.ds(rowbase, CHUNKS)], dstb, insem)

    z16 = jnp.zeros((16,), jnp.float32)

    def _zrow(i, carry):
        zbuf[i, pl.ds(0, 16)] = z16
        zbuf[i, pl.ds(16, 16)] = z16
        return carry

    lax.fori_loop(0, RPT, _zrow, 0)

    @pl.when(sid < NS - 1)
    def _zero_main():
        pltpu.sync_copy(zbuf, acc.at[pl.ds(sid * RPT, RPT)])

    @pl.when(sid == NS - 1)
    def _zero_last():
        pltpu.sync_copy(zbuf.at[pl.ds(0, RLAST)],
                        acc.at[pl.ds((NS - 1) * RPT, RLAST)])

    pltpu.make_async_copy(asrc_hbm, asrc_v, insem).wait()
    pltpu.make_async_copy(adst_hbm, adst_v, insem).wait()
    pltpu.make_async_copy(c_hbm, c_v, insem).wait()
    pltpu.make_async_copy(src_hbm.at[pl.ds(rowbase, CHUNKS)], srcb, insem).wait()
    pltpu.make_async_copy(dst_hbm.at[pl.ds(rowbase, CHUNKS)], dstb, insem).wait()
    plsc.subcore_barrier()

    cvec = c_v[...]
    gbufs = (gbuf0, gbuf1)
    gsems = (gsem0, gsem1)
    ssems = (ssem0, ssem1)

    pltpu.async_copy(hpad_hbm.at[srcb.at[0]], gbuf0, gsem0)

    def _chunk(t, carry):
        for p in range(2):
            @pl.when(lax.rem(t, 2) == p)
            def _body(p=p):
                gb, go = gbufs[p], gbufs[1 - p]
                pltpu.make_async_copy(hpad_hbm.at[srcb.at[t]], gb,
                                      gsems[p]).wait()
                @pl.when(t >= 1)
                def _drain():
                    pltpu.make_async_copy(go, acc.at[dstb.at[t - 1]],
                                          ssems[1 - p]).wait()

                @pl.when(t + 1 < CHUNKS)
                def _prefetch():
                    pltpu.async_copy(hpad_hbm.at[srcb.at[t + 1]], go,
                                     gsems[1 - p])

                for g in range(K // 16):
                    s16 = srcb[t, pl.ds(g * 16, 16)]
                    d16 = dstb[t, pl.ds(g * 16, 16)]
                    e = (plsc.load_gather(asrc_v, [s16])
                         + plsc.load_gather(adst_v, [d16]))
                    e = jnp.where(e >= 0.0, e, 0.2 * e)
                    exb[...] = jnp.exp(e - cvec)
                    for i in range(16):
                        j = g * 16 + i
                        w = plsc.load_gather(exb,
                                             [jnp.full((16,), i, jnp.int32)])
                        gb[j, pl.ds(0, 16)] = gb[j, pl.ds(0, 16)] * w
                        gb[j, pl.ds(16, 16)] = gb[j, pl.ds(16, 16)] * w
                pltpu.async_copy(gb, acc.at[dstb.at[t]], ssems[p], add=True)
        return carry

    lax.fori_loop(0, CHUNKS, _chunk, 0)
    pltpu.make_async_copy(gbufs[(CHUNKS - 1) % 2],
                          acc.at[dstb.at[CHUNKS - 1]],
                          ssems[(CHUNKS - 1) % 2]).wait()
    plsc.subcore_barrier()

    @pl.when(sid < NS - 1)
    def _out_main():
        pltpu.sync_copy(acc.at[pl.ds(sid * RPT, RPT)],
                        out_hbm.at[cid, pl.ds(sid * RPT, RPT)])

    @pl.when(sid == NS - 1)
    def _out_last():
        pltpu.sync_copy(acc.at[pl.ds((NS - 1) * RPT, RLAST)],
                        out_hbm.at[cid, pl.ds((NS - 1) * RPT, RLAST)])


def _finish_body(p_ref, hpad_ref, asrc_ref, adst_ref, c_ref, batch_ref,
                 bgat_ref, outw_ref, outb_ref, o_ref):
    tot = p_ref[0] + p_ref[1]
    e = asrc_ref[...] + adst_ref[...]
    e = jnp.where(e >= 0.0, e, 0.2 * e)
    es = jnp.exp(e - c_ref[0:1, 0:1])
    tot = tot + es * hpad_ref[...]
    node = tot[:, :F_OUT] / (tot[:, F_OUT:F_OUT + 1] + 1e-16) + bgat_ref[...]
    node = jnp.where(node >= 0.0, node, 0.01 * node)
    gids = lax.broadcasted_iota(jnp.int32, (N, G), 1)
    onehot = (batch_ref[...] == gids).astype(jnp.float32)
    sums = lax.dot_general(onehot, node, (((0,), (0,)), ((), ())),
                           preferred_element_type=jnp.float32)
    cnt = lax.dot_general(onehot, jnp.ones((N, 1), jnp.float32),
                          (((0,), (0,)), ((), ())),
                          preferred_element_type=jnp.float32)
    pooled = sums / jnp.maximum(cnt, 1.0)
    logits = jnp.dot(pooled, outw_ref[...],
                     preferred_element_type=jnp.float32) + outb_ref[...]
    mx = jnp.max(logits, axis=1, keepdims=True)
    ex2 = jnp.exp(logits - mx)
    o_ref[...] = ex2 / jnp.sum(ex2, axis=1, keepdims=True)


_finish = pl.pallas_call(
    _finish_body,
    out_shape=jax.ShapeDtypeStruct((G, N_OUT), jnp.float32),
)


def kernel(x, edge_index, edge_attr, batch, W, att_src, att_dst, b_gat,
           out_W, out_b):
    del edge_attr
    hpad, asrc, adst, crow = _prep(x, W, att_src.reshape(1, F_OUT),
                                   att_dst.reshape(1, F_OUT))
    src2d = edge_index[0].reshape(NC * NS * CHUNKS, K)
    dst2d = edge_index[1].reshape(NC * NS * CHUNKS, K)
    partials = _edge_kernel(hpad, src2d, dst2d,
                            asrc.reshape(N), adst.reshape(N), crow[0, :16])
    return _finish(partials, hpad, asrc, adst, crow, batch.reshape(N, 1),
                   b_gat.reshape(1, F_OUT), out_W, out_b)


# speedup vs baseline: 76.1938x; 1.1380x over previous
import functools

import jax
import jax.numpy as jnp
from jax import lax
from jax.experimental import pallas as pl
from jax.experimental.pallas import tpu as pltpu
from jax.experimental.pallas import tpu_sc as plsc

N = 10000
E = 320000
F_IN = 128
F_OUT = 20
G = 64
N_OUT = 5
FP = 32
NC = 2
NS = 16
EPT = E // (NC * NS)
K = 80
CHUNKS = EPT // K
RPT = 632
RLAST = N - (NS - 1) * RPT


def _prep_body(x_ref, w_ref, att2_ref, hpad_ref, anode_ref, c_ref):
    h = jnp.dot(x_ref[...], w_ref[...], preferred_element_type=jnp.float32)
    lane = lax.broadcasted_iota(jnp.int32, (1, FP), 1)
    hpad = h + jnp.where(lane == F_OUT, 1.0, 0.0)
    hpad_ref[...] = hpad
    anode = jnp.dot(hpad, att2_ref[...], preferred_element_type=jnp.float32)
    anode_ref[...] = anode
    m = jnp.max(anode, axis=0, keepdims=True)
    ub = m[0:1, 0:1] + m[0:1, 1:2]
    c = jnp.where(ub >= 0.0, ub, 0.2 * ub)
    c_ref[...] = jnp.broadcast_to(c, (1, 128))


_prep = pl.pallas_call(
    _prep_body,
    out_shape=[
        jax.ShapeDtypeStruct((N, FP), jnp.float32),
        jax.ShapeDtypeStruct((N, 2), jnp.float32),
        jax.ShapeDtypeStruct((1, 128), jnp.float32),
    ],
)


_mesh = plsc.VectorSubcoreMesh(core_axis_name="c", subcore_axis_name="s",
                               num_cores=NC, num_subcores=NS)


@functools.partial(
    pl.kernel,
    mesh=_mesh,
    compiler_params=pltpu.CompilerParams(needs_layout_passes=False,
                                         use_tc_tiling_on_sc=False),
    out_type=jax.ShapeDtypeStruct((NC, N, FP), jnp.float32),
    scratch_types=[
        pltpu.VMEM((2 * N,), jnp.float32),
        pltpu.VMEM((16,), jnp.float32),
        pltpu.VMEM((EPT,), jnp.int32),
        pltpu.VMEM((EPT,), jnp.int32),
        pltpu.VMEM((K, FP), jnp.float32),
        pltpu.VMEM((K, FP), jnp.float32),
        pltpu.VMEM((RPT, FP), jnp.float32),
        pltpu.VMEM_SHARED((N, FP), jnp.float32),
        pltpu.SemaphoreType.DMA,
        pltpu.SemaphoreType.DMA,
        pltpu.SemaphoreType.DMA,
        pltpu.SemaphoreType.DMA,
        pltpu.SemaphoreType.DMA,
    ],
)
def _edge_kernel(hpad_hbm, eidx_hbm, anode_hbm, c_hbm,
                 out_hbm, anode_v, c_v, srcb, dstb, gbuf0, gbuf1,
                 zbuf, acc, insem, gsem0, gsem1, ssem0, ssem1):
    cid = lax.axis_index("c")
    sid = lax.axis_index("s")
    ebase = (cid * NS + sid) * EPT

    pltpu.async_copy(anode_hbm, anode_v, insem)
    pltpu.async_copy(c_hbm, c_v, insem)
    pltpu.async_copy(eidx_hbm.at[0, pl.ds(ebase, EPT)], srcb, insem)
    pltpu.async_copy(eidx_hbm.at[1, pl.ds(ebase, EPT)], dstb, insem)

    z16 = jnp.zeros((16,), jnp.float32)

    def _zrow(i, carry):
        zbuf[i, pl.ds(0, 16)] = z16
        zbuf[i, pl.ds(16, 16)] = z16
        return carry

    lax.fori_loop(0, RPT, _zrow, 0)

    @pl.when(sid < NS - 1)
    def _zero_main():
        pltpu.sync_copy(zbuf, acc.at[pl.ds(sid * RPT, RPT)])

    @pl.when(sid == NS - 1)
    def _zero_last():
        pltpu.sync_copy(zbuf.at[pl.ds(0, RLAST)],
                        acc.at[pl.ds((NS - 1) * RPT, RLAST)])

    pltpu.make_async_copy(anode_hbm, anode_v, insem).wait()
    pltpu.make_async_copy(c_hbm, c_v, insem).wait()
    pltpu.make_async_copy(eidx_hbm.at[0, pl.ds(ebase, EPT)], srcb,
                          insem).wait()
    pltpu.make_async_copy(eidx_hbm.at[1, pl.ds(ebase, EPT)], dstb,
                          insem).wait()
    plsc.subcore_barrier()

    cvec = c_v[...]
    gbufs = (gbuf0, gbuf1)
    gsems = (gsem0, gsem1)
    ssems = (ssem0, ssem1)
    splat_dnums = lax.GatherDimensionNumbers(
        offset_dims=(), collapsed_slice_dims=(0,), start_index_map=(0,))

    pltpu.async_copy(hpad_hbm.at[srcb.at[pl.ds(0, K)]], gbuf0, gsem0)

    def _chunk(t, carry):
        for p in range(2):
            @pl.when(lax.rem(t, 2) == p)
            def _body(p=p):
                gb, go = gbufs[p], gbufs[1 - p]
                pltpu.make_async_copy(hpad_hbm.at[srcb.at[pl.ds(t * K, K)]],
                                      gb, gsems[p]).wait()
                @pl.when(t >= 1)
                def _drain():
                    pltpu.make_async_copy(
                        go, acc.at[dstb.at[pl.ds((t - 1) * K, K)]],
                        ssems[1 - p]).wait()

                @pl.when(t + 1 < CHUNKS)
                def _prefetch():
                    pltpu.async_copy(
                        hpad_hbm.at[srcb.at[pl.ds((t + 1) * K, K)]],
                        go, gsems[1 - p])

                for g in range(K // 16):
                    s16 = srcb[pl.ds(t * K + g * 16, 16)]
                    d16 = dstb[pl.ds(t * K + g * 16, 16)]
                    e = (plsc.load_gather(anode_v, [s16 * 2])
                         + plsc.load_gather(anode_v, [d16 * 2 + 1]))
                    e = jnp.where(e >= 0.0, e, 0.2 * e)
                    ex = jnp.exp(e - cvec)
                    for i in range(16):
                        j = g * 16 + i
                        w = lax.gather(
                            ex, jnp.full((16, 1), i, jnp.int32), splat_dnums,
                            (1,),
                            mode=lax.GatherScatterMode.PROMISE_IN_BOUNDS)
                        gb[j, pl.ds(0, 16)] = gb[j, pl.ds(0, 16)] * w
                        gb[j, pl.ds(16, 16)] = gb[j, pl.ds(16, 16)] * w
                pltpu.async_copy(gb, acc.at[dstb.at[pl.ds(t * K, K)]],
                                 ssems[p], add=True)
        return carry

    lax.fori_loop(0, CHUNKS, _chunk, 0)
    pltpu.make_async_copy(gbufs[(CHUNKS - 1) % 2],
                          acc.at[dstb.at[pl.ds((CHUNKS - 1) * K, K)]],
                          ssems[(CHUNKS - 1) % 2]).wait()
    plsc.subcore_barrier()

    @pl.when(sid < NS - 1)
    def _out_main():
        pltpu.sync_copy(acc.at[pl.ds(sid * RPT, RPT)],
                        out_hbm.at[cid, pl.ds(sid * RPT, RPT)])

    @pl.when(sid == NS - 1)
    def _out_last():
        pltpu.sync_copy(acc.at[pl.ds((NS - 1) * RPT, RLAST)],
                        out_hbm.at[cid, pl.ds((NS - 1) * RPT, RLAST)])


def _finish_body(p_ref, hpad_ref, anode_ref, c_ref, batch_ref,
                 bgat_ref, outw_ref, outb_ref, o_ref):
    tot = p_ref[0] + p_ref[1]
    e = anode_ref[:, 0:1] + anode_ref[:, 1:2]
    e = jnp.where(e >= 0.0, e, 0.2 * e)
    es = jnp.exp(e - c_ref[0:1, 0:1])
    tot = tot + es * hpad_ref[...]
    node = tot[:, :F_OUT] / (tot[:, F_OUT:F_OUT + 1] + 1e-16) + bgat_ref[...]
    node = jnp.where(node >= 0.0, node, 0.01 * node)
    gids = lax.broadcasted_iota(jnp.int32, (G, N), 0)
    onehot = (batch_ref[...] == gids).astype(jnp.float32)
    sums = lax.dot_general(onehot, node, (((1,), (0,)), ((), ())),
                           preferred_element_type=jnp.float32)
    cnt = lax.dot_general(onehot, jnp.ones((N, 1), jnp.float32),
                          (((1,), (0,)), ((), ())),
                          preferred_element_type=jnp.float32)
    pooled = sums / jnp.maximum(cnt, 1.0)
    logits = jnp.dot(pooled, outw_ref[...],
                     preferred_element_type=jnp.float32) + outb_ref[...]
    mx = jnp.max(logits, axis=1, keepdims=True)
    ex2 = jnp.exp(logits - mx)
    o_ref[...] = ex2 / jnp.sum(ex2, axis=1, keepdims=True)


_finish = pl.pallas_call(
    _finish_body,
    out_shape=jax.ShapeDtypeStruct((G, N_OUT), jnp.float32),
)


def kernel(x, edge_index, edge_attr, batch, W, att_src, att_dst, b_gat,
           out_W, out_b):
    del edge_attr
    w_pad = jnp.concatenate(
        [W, jnp.zeros((F_IN, FP - F_OUT), jnp.float32)], axis=1)
    att2 = jnp.stack(
        [jnp.concatenate([att_src, jnp.zeros((FP - F_OUT,), jnp.float32)]),
         jnp.concatenate([att_dst, jnp.zeros((FP - F_OUT,), jnp.float32)])],
        axis=1)
    hpad, anode, crow = _prep(x, w_pad, att2)
    partials = _edge_kernel(hpad, edge_index, anode.reshape(2 * N),
                            crow[0, :16])
    return _finish(partials, hpad, anode, crow, batch.reshape(1, N),
                   b_gat.reshape(1, F_OUT), out_W, out_b)


# speedup vs baseline: 76.2273x; 1.0004x over previous
import functools

import jax
import jax.numpy as jnp
from jax import lax
from jax.experimental import pallas as pl
from jax.experimental.pallas import tpu as pltpu
from jax.experimental.pallas import tpu_sc as plsc

N = 10000
E = 320000
F_IN = 128
F_OUT = 20
G = 64
N_OUT = 5
FP = 32
NC = 2
NS = 16
EPT = E // (NC * NS)
K = 80
CHUNKS = EPT // K
RPT = 632
RLAST = N - (NS - 1) * RPT


def _prep_body(x_ref, w_ref, att2_ref, hpad_ref, anode_ref, c_ref):
    h = jnp.dot(x_ref[...], w_ref[...], preferred_element_type=jnp.float32)
    lane = lax.broadcasted_iota(jnp.int32, (1, FP), 1)
    hpad = h + jnp.where(lane == F_OUT, 1.0, 0.0)
    hpad_ref[...] = hpad
    anode = jnp.dot(hpad, att2_ref[...], preferred_element_type=jnp.float32)
    anode_ref[...] = anode
    m = jnp.max(anode, axis=0, keepdims=True)
    ub = m[0:1, 0:1] + m[0:1, 1:2]
    c = jnp.where(ub >= 0.0, ub, 0.2 * ub)
    c_ref[...] = jnp.broadcast_to(c, (1, 128))


_prep = pl.pallas_call(
    _prep_body,
    out_shape=[
        jax.ShapeDtypeStruct((N, FP), jnp.float32),
        jax.ShapeDtypeStruct((N, 2), jnp.float32),
        jax.ShapeDtypeStruct((1, 128), jnp.float32),
    ],
)


_mesh = plsc.VectorSubcoreMesh(core_axis_name="c", subcore_axis_name="s",
                               num_cores=NC, num_subcores=NS)


@functools.partial(
    pl.kernel,
    mesh=_mesh,
    compiler_params=pltpu.CompilerParams(needs_layout_passes=False,
                                         use_tc_tiling_on_sc=False),
    out_type=jax.ShapeDtypeStruct((NC, N, FP), jnp.float32),
    scratch_types=[
        pltpu.VMEM((2 * N,), jnp.float32),
        pltpu.VMEM((16,), jnp.float32),
        pltpu.VMEM((EPT,), jnp.int32),
        pltpu.VMEM((EPT,), jnp.int32),
        pltpu.VMEM((K, FP), jnp.float32),
        pltpu.VMEM((K, FP), jnp.float32),
        pltpu.VMEM((K,), jnp.float32),
        pltpu.VMEM((RPT, FP), jnp.float32),
        pltpu.VMEM_SHARED((N, FP), jnp.float32),
        pltpu.SemaphoreType.DMA,
        pltpu.SemaphoreType.DMA,
        pltpu.SemaphoreType.DMA,
        pltpu.SemaphoreType.DMA,
        pltpu.SemaphoreType.DMA,
    ],
)
def _edge_kernel(hpad_hbm, eidx_hbm, anode_hbm, c_hbm,
                 out_hbm, anode_v, c_v, srcb, dstb, gbuf0, gbuf1, exbuf,
                 zbuf, acc, insem, gsem0, gsem1, ssem0, ssem1):
    cid = lax.axis_index("c")
    sid = lax.axis_index("s")
    ebase = (cid * NS + sid) * EPT

    pltpu.async_copy(anode_hbm, anode_v, insem)
    pltpu.async_copy(c_hbm, c_v, insem)
    pltpu.async_copy(eidx_hbm.at[0, pl.ds(ebase, EPT)], srcb, insem)
    pltpu.async_copy(eidx_hbm.at[1, pl.ds(ebase, EPT)], dstb, insem)

    z16 = jnp.zeros((16,), jnp.float32)

    def _zrow(i, carry):
        zbuf[i, pl.ds(0, 16)] = z16
        zbuf[i, pl.ds(16, 16)] = z16
        return carry

    lax.fori_loop(0, RPT, _zrow, 0)

    @pl.when(sid < NS - 1)
    def _zero_main():
        pltpu.sync_copy(zbuf, acc.at[pl.ds(sid * RPT, RPT)])

    @pl.when(sid == NS - 1)
    def _zero_last():
        pltpu.sync_copy(zbuf.at[pl.ds(0, RLAST)],
                        acc.at[pl.ds((NS - 1) * RPT, RLAST)])

    pltpu.make_async_copy(anode_hbm, anode_v, insem).wait()
    pltpu.make_async_copy(c_hbm, c_v, insem).wait()
    pltpu.make_async_copy(eidx_hbm.at[0, pl.ds(ebase, EPT)], srcb,
                          insem).wait()
    pltpu.make_async_copy(eidx_hbm.at[1, pl.ds(ebase, EPT)], dstb,
                          insem).wait()
    plsc.subcore_barrier()

    cvec = c_v[...]
    gbufs = (gbuf0, gbuf1)
    gsems = (gsem0, gsem1)
    ssems = (ssem0, ssem1)
    pltpu.async_copy(hpad_hbm.at[srcb.at[pl.ds(0, K)]], gbuf0, gsem0)

    def _chunk(t, carry):
        for p in range(2):
            @pl.when(lax.rem(t, 2) == p)
            def _body(p=p):
                gb, go = gbufs[p], gbufs[1 - p]
                pltpu.make_async_copy(hpad_hbm.at[srcb.at[pl.ds(t * K, K)]],
                                      gb, gsems[p]).wait()
                @pl.when(t >= 1)
                def _drain():
                    pltpu.make_async_copy(
                        go, acc.at[dstb.at[pl.ds((t - 1) * K, K)]],
                        ssems[1 - p]).wait()

                @pl.when(t + 1 < CHUNKS)
                def _prefetch():
                    pltpu.async_copy(
                        hpad_hbm.at[srcb.at[pl.ds((t + 1) * K, K)]],
                        go, gsems[1 - p])

                @plsc.parallel_loop(0, K, step=16, unroll=2)
                def _exg(o):
                    s16 = srcb[pl.ds(t * K + o, 16)]
                    d16 = dstb[pl.ds(t * K + o, 16)]
                    e = (plsc.load_gather(anode_v, [s16 * 2])
                         + plsc.load_gather(anode_v, [d16 * 2 + 1]))
                    e = jnp.where(e >= 0.0, e, 0.2 * e)
                    exbuf[pl.ds(o, 16)] = jnp.exp(e - cvec)

                @plsc.parallel_loop(0, K, step=1, unroll=8)
                def _scale(j):
                    w = plsc.load_gather(exbuf,
                                         [jnp.full((16,), j, jnp.int32)])
                    gb[j, pl.ds(0, 16)] = gb[j, pl.ds(0, 16)] * w
                    gb[j, pl.ds(16, 16)] = gb[j, pl.ds(16, 16)] * w
                pltpu.async_copy(gb, acc.at[dstb.at[pl.ds(t * K, K)]],
                                 ssems[p], add=True)
        return carry

    lax.fori_loop(0, CHUNKS, _chunk, 0)
    pltpu.make_async_copy(gbufs[(CHUNKS - 1) % 2],
                          acc.at[dstb.at[pl.ds((CHUNKS - 1) * K, K)]],
                          ssems[(CHUNKS - 1) % 2]).wait()
    plsc.subcore_barrier()

    @pl.when(sid < NS - 1)
    def _out_main():
        pltpu.sync_copy(acc.at[pl.ds(sid * RPT, RPT)],
                        out_hbm.at[cid, pl.ds(sid * RPT, RPT)])

    @pl.when(sid == NS - 1)
    def _out_last():
        pltpu.sync_copy(acc.at[pl.ds((NS - 1) * RPT, RLAST)],
                        out_hbm.at[cid, pl.ds((NS - 1) * RPT, RLAST)])


def _finish_body(p_ref, hpad_ref, anode_ref, c_ref, batch_ref,
                 bgat_ref, outw_ref, outb_ref, o_ref):
    tot = p_ref[0] + p_ref[1]
    e = anode_ref[:, 0:1] + anode_ref[:, 1:2]
    e = jnp.where(e >= 0.0, e, 0.2 * e)
    es = jnp.exp(e - c_ref[0:1, 0:1])
    tot = tot + es * hpad_ref[...]
    node = tot[:, :F_OUT] / (tot[:, F_OUT:F_OUT + 1] + 1e-16) + bgat_ref[...]
    node = jnp.where(node >= 0.0, node, 0.01 * node)
    gids = lax.broadcasted_iota(jnp.int32, (G, N), 0)
    onehot = (batch_ref[...] == gids).astype(jnp.float32)
    sums = lax.dot_general(onehot, node, (((1,), (0,)), ((), ())),
                           preferred_element_type=jnp.float32)
    cnt = lax.dot_general(onehot, jnp.ones((N, 1), jnp.float32),
                          (((1,), (0,)), ((), ())),
                          preferred_element_type=jnp.float32)
    pooled = sums / jnp.maximum(cnt, 1.0)
    logits = jnp.dot(pooled, outw_ref[...],
                     preferred_element_type=jnp.float32) + outb_ref[...]
    mx = jnp.max(logits, axis=1, keepdims=True)
    ex2 = jnp.exp(logits - mx)
    o_ref[...] = ex2 / jnp.sum(ex2, axis=1, keepdims=True)


_finish = pl.pallas_call(
    _finish_body,
    out_shape=jax.ShapeDtypeStruct((G, N_OUT), jnp.float32),
)


def kernel(x, edge_index, edge_attr, batch, W, att_src, att_dst, b_gat,
           out_W, out_b):
    del edge_attr
    w_pad = jnp.concatenate(
        [W, jnp.zeros((F_IN, FP - F_OUT), jnp.float32)], axis=1)
    att2 = jnp.stack(
        [jnp.concatenate([att_src, jnp.zeros((FP - F_OUT,), jnp.float32)]),
         jnp.concatenate([att_dst, jnp.zeros((FP - F_OUT,), jnp.float32)])],
        axis=1)
    hpad, anode, crow = _prep(x, w_pad, att2)
    partials = _edge_kernel(hpad, edge_index, anode.reshape(2 * N),
                            crow[0, :16])
    return _finish(partials, hpad, anode, crow, batch.reshape(1, N),
                   b_gat.reshape(1, F_OUT), out_W, out_b)
